# baseline (speedup 1.0000x reference)
import functools

import jax
import jax.numpy as jnp
from jax import lax
from jax.experimental import pallas as pl
from jax.experimental.pallas import tpu as pltpu
from jax.experimental.pallas import tpu_sc as plsc

B, N, D = 4, 576, 384
LAMBDA_SPATIAL = 0.5
SIGMA_SQ = 100.0 * 100.0
NEG = -3.0e38
L = 16
NCHUNK = N // L
ROWS_PER_TILE = N // 8


def _tc_body(feat_ref, coord_ref, coordT_ref,
             sim_out, pen_out, row0_out, startv_out):
    iota_l = lax.broadcasted_iota(jnp.int32, (1, N), 1)
    for b in range(B):
        f = feat_ref[b]
        s = jnp.sum(f * f, axis=1, keepdims=True)
        norm = jnp.maximum(jnp.sqrt(s), 1e-12)
        fn = f / norm
        sem = lax.dot_general(fn, fn, (((1,), (1,)), ((), ())),
                              preferred_element_type=jnp.float32)
        x_col = coord_ref[b][:, 0:1]
        y_col = coord_ref[b][:, 1:2]
        x_row = coordT_ref[b][0:1, :]
        y_row = coordT_ref[b][1:2, :]
        dx = x_col - x_row
        dy = y_col - y_row
        dist = jnp.sqrt(dx * dx + dy * dy)
        spat = jnp.exp(-(dist * dist) / SIGMA_SQ)
        sim_out[b] = sem + LAMBDA_SPATIAL * spat

        conn = jnp.sum(sim_out[b], axis=1, keepdims=True)
        mconn = jnp.max(conn)
        iota_s = lax.broadcasted_iota(jnp.int32, (N, 1), 0)
        start = jnp.min(jnp.where(conn == mconn, iota_s, N))
        pen_out[pl.ds(b, 1), :] = jnp.where(iota_l == start, NEG, 0.0)
        row0_out[pl.ds(b, 1), :] = sim_out[b, pl.ds(start, 1), :]
        startv_out[pl.ds(b, 1), :] = jnp.full((1, N), start,
                                              jnp.int32).astype(jnp.float32)


def _tc_stage(features, coords, coordT):
    return pl.pallas_call(
        _tc_body,
        out_shape=(
            jax.ShapeDtypeStruct((B, N, N), jnp.float32),
            jax.ShapeDtypeStruct((B, N), jnp.float32),
            jax.ShapeDtypeStruct((B, N), jnp.float32),
            jax.ShapeDtypeStruct((B, N), jnp.float32),
        ),
    )(features, coords, coordT)


def _argmax_chunks(vals, idxs):
    while len(vals) > 1:
        nv, ni = [], []
        for k in range(0, len(vals) - 1, 2):
            take_hi = vals[k + 1] > vals[k]
            nv.append(jnp.where(take_hi, vals[k + 1], vals[k]))
            ni.append(jnp.where(take_hi, idxs[k + 1], idxs[k]))
        if len(vals) % 2:
            nv.append(vals[-1])
            ni.append(idxs[-1])
        vals, idxs = nv, ni
    return vals[0], idxs[0]


def _store1(ref, pos, val, lane_i):
    off = pl.multiple_of((pos >> 4) * L, L)
    chunk = ref[pl.ds(off, L)]
    ref[pl.ds(off, L)] = jnp.where(lane_i == (pos & (L - 1)), val, chunk)


def _sc_body(sim_hbm, pen_hbm, row0_hbm, startv_hbm, feat_hbm,
             out_r_hbm, out_o_hbm, ord2_hbm,
             spmem_sim, row_v, pen_v, ord_v, ord2_v, st_v,
             idx_v, rows_v, sem, sem2):
    c = lax.axis_index("c")
    s = lax.axis_index("s")

    for bl in range(2):
        gbase = (2 * c + bl) * (N * N)
        lbase = bl * (N * N)
        pltpu.sync_copy(
            sim_hbm.at[pl.ds(gbase + s * NCHUNK * N, NCHUNK * N)],
            spmem_sim.at[pl.ds(lbase + s * NCHUNK * N, NCHUNK * N)])
    plsc.subcore_barrier()

    @pl.when(s < 2)
    def _greedy():
        b = 2 * c + s
        boff = b * N
        lbase = s * (N * N)
        pltpu.sync_copy(row0_hbm.at[pl.ds(boff, N)], row_v)
        pltpu.sync_copy(pen_hbm.at[pl.ds(boff, N)], pen_v)
        pltpu.sync_copy(startv_hbm.at[pl.ds(boff, L)], st_v)
        start = st_v[...][0].astype(jnp.int32)
        lane_i = lax.broadcasted_iota(jnp.int32, (L,), 0)
        lane = lane_i.astype(jnp.float32)
        ord_v[pl.ds(0, L)] = jnp.where(lane_i == 0, start, ord_v[pl.ds(0, L)])
        ord2_v[pl.ds(0, L)] = jnp.where(lane_i == 0, start + boff,
                                        ord2_v[pl.ds(0, L)])
        pen0 = tuple(pen_v[pl.ds(ch * L, L)] for ch in range(NCHUNK))

        def step(i, carry):
            pen = carry[1:]
            vals, idxs = [], []
            for ch in range(NCHUNK):
                v = row_v[pl.ds(ch * L, L)] + pen[ch]
                vals.append(v)
                idxs.append(lane + float(ch * L))
            val, idx = _argmax_chunks(vals, idxs)
            for k in (8, 4, 2, 1):
                perm = jnp.bitwise_xor(lane_i, k)
                vs = val.at[perm].get(mode="promise_in_bounds")
                ids = idx.at[perm].get(mode="promise_in_bounds")
                swap = (vs > val) | ((vs == val) & (ids < idx))
                val = jnp.where(swap, vs, val)
                idx = jnp.where(swap, ids, idx)
            nxt_f = idx[0]
            nxt = nxt_f.astype(jnp.int32)
            off = pl.multiple_of(lbase + nxt * N, 8)
            dma = pltpu.async_copy(spmem_sim.at[pl.ds(off, N)], row_v, sem2)
            _store1(ord_v, i, nxt, lane_i)
            _store1(ord2_v, i, nxt + boff, lane_i)
            pen = tuple(
                jnp.where(idxs[ch] == nxt_f, NEG, pen[ch])
                for ch in range(NCHUNK))
            dma.wait()
            return (nxt,) + pen

        lax.fori_loop(1, N, step, (start,) + pen0, unroll=False)
        pltpu.sync_copy(ord_v, out_o_hbm.at[pl.ds(boff, N)])
        pltpu.sync_copy(ord2_v, ord2_hbm.at[pl.ds(boff, N)])

    plsc.subcore_barrier()

    bg = 2 * c + s // 8
    r0 = bg * N + (s % 8) * ROWS_PER_TILE
    pltpu.sync_copy(ord2_hbm.at[pl.ds(r0, ROWS_PER_TILE)], idx_v)
    pltpu.async_copy(feat_hbm.at[idx_v], rows_v, sem).wait()
    pltpu.sync_copy(rows_v, out_r_hbm.at[pl.ds(r0, ROWS_PER_TILE)])


_sc_kernel = functools.partial(
    pl.kernel,
    out_type=(
        jax.ShapeDtypeStruct((B * N, D), jnp.float32),
        jax.ShapeDtypeStruct((B * N,), jnp.int32),
        jax.ShapeDtypeStruct((B * N,), jnp.int32),
    ),
    mesh=plsc.VectorSubcoreMesh(core_axis_name="c", subcore_axis_name="s",
                                num_cores=2, num_subcores=16),
    scratch_types=[
        pltpu.VMEM_SHARED((2 * N * N,), jnp.float32),
        pltpu.VMEM((N,), jnp.float32),
        pltpu.VMEM((N,), jnp.float32),
        pltpu.VMEM((N,), jnp.int32),
        pltpu.VMEM((N,), jnp.int32),
        pltpu.VMEM((L,), jnp.float32),
        pltpu.VMEM((ROWS_PER_TILE,), jnp.int32),
        pltpu.VMEM((ROWS_PER_TILE, D), jnp.float32),
        pltpu.SemaphoreType.DMA,
        pltpu.SemaphoreType.DMA,
    ],
)(_sc_body)


def kernel(features, coords):
    coordT = jnp.swapaxes(coords, 1, 2)
    sim, pen0, row0, startv = _tc_stage(features, coords, coordT)
    feat2 = features.reshape(B * N, D)
    reordered2, orders1, _ = _sc_kernel(
        sim.reshape(-1), pen0.reshape(-1), row0.reshape(-1),
        startv.reshape(-1), feat2)
    return reordered2.reshape(B, N, D), orders1.reshape(B, N)


# speedup vs baseline: 1.0258x; 1.0258x over previous
import functools

import jax
import jax.numpy as jnp
from jax import lax
from jax.experimental import pallas as pl
from jax.experimental.pallas import tpu as pltpu
from jax.experimental.pallas import tpu_sc as plsc

B, N, D = 4, 576, 384
LAMBDA_SPATIAL = 0.5
SIGMA_SQ = 100.0 * 100.0
NEG = -3.0e38
L = 16
NCHUNK = N // L
ROWS_PER_TILE = N // 8


def _tc_body(feat_ref, coord_ref, coordT_ref,
             sim_out, pen_out, row0_out, startv_out):
    iota_l = lax.broadcasted_iota(jnp.int32, (1, N), 1)
    for b in range(B):
        f = feat_ref[b]
        s = jnp.sum(f * f, axis=1, keepdims=True)
        norm = jnp.maximum(jnp.sqrt(s), 1e-12)
        fn = f / norm
        sem = lax.dot_general(fn, fn, (((1,), (1,)), ((), ())),
                              preferred_element_type=jnp.float32)
        x_col = coord_ref[b][:, 0:1]
        y_col = coord_ref[b][:, 1:2]
        x_row = coordT_ref[b][0:1, :]
        y_row = coordT_ref[b][1:2, :]
        dx = x_col - x_row
        dy = y_col - y_row
        dist = jnp.sqrt(dx * dx + dy * dy)
        spat = jnp.exp(-(dist * dist) / SIGMA_SQ)
        sim_out[b] = sem + LAMBDA_SPATIAL * spat

        conn = jnp.sum(sim_out[b], axis=1, keepdims=True)
        mconn = jnp.max(conn)
        iota_s = lax.broadcasted_iota(jnp.int32, (N, 1), 0)
        start = jnp.min(jnp.where(conn == mconn, iota_s, N))
        pen_out[pl.ds(b, 1), :] = jnp.where(iota_l == start, NEG, 0.0)
        row0_out[pl.ds(b, 1), :] = sim_out[b, pl.ds(start, 1), :]
        startv_out[pl.ds(b, 1), :] = jnp.full((1, N), start,
                                              jnp.int32).astype(jnp.float32)


def _tc_stage(features, coords, coordT):
    return pl.pallas_call(
        _tc_body,
        out_shape=(
            jax.ShapeDtypeStruct((B, N, N), jnp.float32),
            jax.ShapeDtypeStruct((B, N), jnp.float32),
            jax.ShapeDtypeStruct((B, N), jnp.float32),
            jax.ShapeDtypeStruct((B, N), jnp.float32),
        ),
    )(features, coords, coordT)


def _argmax_chunks(vals, idxs):
    while len(vals) > 1:
        nv, ni = [], []
        for k in range(0, len(vals) - 1, 2):
            take_hi = vals[k + 1] > vals[k]
            nv.append(jnp.where(take_hi, vals[k + 1], vals[k]))
            ni.append(jnp.where(take_hi, idxs[k + 1], idxs[k]))
        if len(vals) % 2:
            nv.append(vals[-1])
            ni.append(idxs[-1])
        vals, idxs = nv, ni
    return vals[0], idxs[0]


def _store1(ref, pos, val, lane_i):
    off = pl.multiple_of((pos >> 4) * L, L)
    chunk = ref[pl.ds(off, L)]
    ref[pl.ds(off, L)] = jnp.where(lane_i == (pos & (L - 1)), val, chunk)


def _sc_body(sim_hbm, pen_hbm, row0_hbm, startv_hbm, feat_hbm,
             out_r_hbm, out_o_hbm, ord2_hbm,
             spmem_sim, row_v, pen_v, ord_v, ord2_v, st_v,
             idx_v, rows_v, sem, sem2, sem3):
    c = lax.axis_index("c")
    s = lax.axis_index("s")

    for bl in range(2):
        gbase = (2 * c + bl) * (N * N)
        lbase = bl * (N * N)
        pltpu.sync_copy(
            sim_hbm.at[pl.ds(gbase + s * NCHUNK * N, NCHUNK * N)],
            spmem_sim.at[pl.ds(lbase + s * NCHUNK * N, NCHUNK * N)])
    plsc.subcore_barrier()

    @pl.when(s < 2)
    def _greedy():
        b = 2 * c + s
        boff = b * N
        lbase = s * (N * N)
        pltpu.sync_copy(row0_hbm.at[pl.ds(boff, N)], row_v)
        pltpu.sync_copy(pen_hbm.at[pl.ds(boff, N)], pen_v)
        pltpu.sync_copy(startv_hbm.at[pl.ds(boff, L)], st_v)
        start = st_v[...][0].astype(jnp.int32)
        lane_i = lax.broadcasted_iota(jnp.int32, (L,), 0)
        lane = lane_i.astype(jnp.float32)
        ord_v[pl.ds(0, L)] = jnp.where(lane_i == 0, start, ord_v[pl.ds(0, L)])
        ord2_v[pl.ds(0, L)] = jnp.where(lane_i == 0, start + boff,
                                        ord2_v[pl.ds(0, L)])
        pen0 = tuple(pen_v[pl.ds(ch * L, L)] for ch in range(NCHUNK))

        half = NCHUNK // 2
        off0 = pl.multiple_of(lbase + start * N + half * L, 8)
        pltpu.async_copy(spmem_sim.at[pl.ds(off0, N - half * L)],
                         row_v.at[pl.ds(half * L, N - half * L)], sem3)

        def step(i, carry):
            pen = carry[1:]
            idxs = [lane + float(ch * L) for ch in range(NCHUNK)]
            valsA = [row_v[pl.ds(ch * L, L)] + pen[ch] for ch in range(half)]
            valA, idxA = _argmax_chunks(valsA, idxs[:half])
            pltpu.make_async_copy(
                row0_hbm.at[pl.ds(0, N - half * L)],
                row_v.at[pl.ds(half * L, N - half * L)], sem3).wait()
            valsB = [row_v[pl.ds(ch * L, L)] + pen[ch]
                     for ch in range(half, NCHUNK)]
            valB, idxB = _argmax_chunks(valsB, idxs[half:])
            val, idx = _argmax_chunks([valA, valB], [idxA, idxB])
            for k in (8, 4, 2, 1):
                perm = jnp.bitwise_xor(lane_i, k)
                vs = val.at[perm].get(mode="promise_in_bounds")
                ids = idx.at[perm].get(mode="promise_in_bounds")
                swap = (vs > val) | ((vs == val) & (ids < idx))
                val = jnp.where(swap, vs, val)
                idx = jnp.where(swap, ids, idx)
            nxt_f = idx[0]
            nxt = nxt_f.astype(jnp.int32)
            off = pl.multiple_of(lbase + nxt * N, 8)
            dmaA = pltpu.async_copy(
                spmem_sim.at[pl.ds(off, half * L)],
                row_v.at[pl.ds(0, half * L)], sem2)
            pltpu.async_copy(
                spmem_sim.at[pl.ds(off + half * L, N - half * L)],
                row_v.at[pl.ds(half * L, N - half * L)], sem3)
            _store1(ord_v, i, nxt, lane_i)
            _store1(ord2_v, i, nxt + boff, lane_i)
            pen = tuple(
                jnp.where(idxs[ch] == nxt_f, NEG, pen[ch])
                for ch in range(NCHUNK))
            dmaA.wait()
            return (nxt,) + pen

        lax.fori_loop(1, N, step, (start,) + pen0, unroll=False)
        pltpu.make_async_copy(
            row0_hbm.at[pl.ds(0, N - half * L)],
            row_v.at[pl.ds(half * L, N - half * L)], sem3).wait()
        pltpu.sync_copy(ord_v, out_o_hbm.at[pl.ds(boff, N)])
        pltpu.sync_copy(ord2_v, ord2_hbm.at[pl.ds(boff, N)])

    plsc.subcore_barrier()

    bg = 2 * c + s // 8
    r0 = bg * N + (s % 8) * ROWS_PER_TILE
    pltpu.sync_copy(ord2_hbm.at[pl.ds(r0, ROWS_PER_TILE)], idx_v)
    pltpu.async_copy(feat_hbm.at[idx_v], rows_v, sem).wait()
    pltpu.sync_copy(rows_v, out_r_hbm.at[pl.ds(r0, ROWS_PER_TILE)])


_sc_kernel = functools.partial(
    pl.kernel,
    out_type=(
        jax.ShapeDtypeStruct((B * N, D), jnp.float32),
        jax.ShapeDtypeStruct((B * N,), jnp.int32),
        jax.ShapeDtypeStruct((B * N,), jnp.int32),
    ),
    mesh=plsc.VectorSubcoreMesh(core_axis_name="c", subcore_axis_name="s",
                                num_cores=2, num_subcores=16),
    scratch_types=[
        pltpu.VMEM_SHARED((2 * N * N,), jnp.float32),
        pltpu.VMEM((N,), jnp.float32),
        pltpu.VMEM((N,), jnp.float32),
        pltpu.VMEM((N,), jnp.int32),
        pltpu.VMEM((N,), jnp.int32),
        pltpu.VMEM((L,), jnp.float32),
        pltpu.VMEM((ROWS_PER_TILE,), jnp.int32),
        pltpu.VMEM((ROWS_PER_TILE, D), jnp.float32),
        pltpu.SemaphoreType.DMA,
        pltpu.SemaphoreType.DMA,
        pltpu.SemaphoreType.DMA,
    ],
)(_sc_body)


def kernel(features, coords):
    coordT = jnp.swapaxes(coords, 1, 2)
    sim, pen0, row0, startv = _tc_stage(features, coords, coordT)
    feat2 = features.reshape(B * N, D)
    reordered2, orders1, _ = _sc_kernel(
        sim.reshape(-1), pen0.reshape(-1), row0.reshape(-1),
        startv.reshape(-1), feat2)
    return reordered2.reshape(B, N, D), orders1.reshape(B, N)
